# baseline (speedup 1.0000x reference)
import jax
import jax.numpy as jnp
from jax import lax
from jax.experimental import pallas as pl
from jax.experimental.pallas import tpu as pltpu
from jax.experimental.pallas import tpu_sc as plsc

_N_EMB = 26
_VOCAB = 100000
_EMB_DIM = 32
_B = 1024
_L = 50
_N_NUM = 2

_EMB_COLS = _N_EMB * _EMB_DIM
_D_OUT = _EMB_COLS + _N_NUM

_NC = 2
_NS = 16
_NW = _NC * _NS
_B_PER_W = _B // _NW

_VCH = 2000
_N_UNITS = _N_EMB * (_VOCAB // _VCH)
_UNIT_ITERS = -(-_N_UNITS // _NW)


def _conv_body(tabt_hbm, out_hbm, slab_v, flat_v, sem):
    wid = lax.axis_index("s") * _NC + lax.axis_index("c")
    lane = jax.lax.iota(jnp.int32, 16)

    @pl.loop(0, _UNIT_ITERS)
    def _unit(u):
        uid = u * _NW + wid

        @pl.when(uid < _N_UNITS)
        def _go():
            e = uid // (_VOCAB // _VCH)
            vc = uid % (_VOCAB // _VCH)
            pltpu.sync_copy(tabt_hbm.at[e, :, pl.ds(vc * _VCH, _VCH)], slab_v)

            @pl.loop(0, _VCH // 16)
            def _t(j):
                rows = (lane + j * 16) * _EMB_DIM
                for d in range(_EMB_DIM):
                    v = slab_v[d, pl.ds(j * 16, 16)]
                    plsc.store_scatter(flat_v, [rows + d], v)

            pltpu.async_copy(
                flat_v,
                out_hbm.at[pl.ds((e * _VOCAB + vc * _VCH) * _EMB_DIM,
                                 _VCH * _EMB_DIM)],
                sem,
            ).wait()


@jax.jit
def _sc_convert(tabt):
    mesh = plsc.VectorSubcoreMesh(core_axis_name="c", subcore_axis_name="s")
    return pl.kernel(
        _conv_body,
        out_type=jax.ShapeDtypeStruct((_N_EMB * _VOCAB * _EMB_DIM,),
                                      jnp.float32),
        mesh=mesh,
        scratch_types=[
            pltpu.VMEM((_EMB_DIM, _VCH), jnp.float32),
            pltpu.VMEM((_VCH * _EMB_DIM,), jnp.float32),
            pltpu.SemaphoreType.DMA,
        ],
        compiler_params=pltpu.CompilerParams(
            use_tc_tiling_on_sc=False, needs_layout_passes=False
        ),
    )(tabt)


def _body(idx_hbm, num_hbm, tab_hbm, out_hbm, idx_v, num_v, np_v, g3, sem, wsem):
    wid = lax.axis_index("s") * _NC + lax.axis_index("c")
    b0 = wid * _B_PER_W
    lane = jax.lax.iota(jnp.int32, 16)

    for e in range(_N_EMB):
        pltpu.sync_copy(idx_hbm.at[e, pl.ds(b0, _B_PER_W)], idx_v.at[e])
    for n in range(_N_NUM):
        pltpu.sync_copy(num_hbm.at[n, pl.ds(b0, _B_PER_W)], num_v.at[n])

    for e in range(1, _N_EMB):
        off = jnp.full((16,), e * _VOCAB, jnp.int32)

        @pl.loop(0, _B_PER_W)
        def _add(bb, e=e, off=off):
            for j in range(_L // 16):
                sl = pl.ds(j * 16, 16)
                idx_v[e, bb, sl] = idx_v[e, bb, sl] + off
            tl = pl.ds(_L - 16, 16)
            tmask = lane >= 16 - (_L % 16)
            v = plsc.load_gather(idx_v, [jnp.full((16,), e, jnp.int32),
                                         jnp.full((16,), bb, jnp.int32),
                                         lane + (_L - 16)], mask=tmask)
            plsc.store_scatter(idx_v, [jnp.full((16,), e, jnp.int32),
                                       jnp.full((16,), bb, jnp.int32),
                                       lane + (_L - 16)], v + off, mask=tmask)

    @pl.loop(0, _B_PER_W)
    def _nt(bb):
        bcol = jnp.full((16,), 0, jnp.int32) + bb
        for n in range(_N_NUM):
            ncol = jnp.full((16,), n, jnp.int32)
            for j in range(_L // 16):
                v = num_v[n, bb, pl.ds(j * 16, 16)]
                plsc.store_scatter(np_v, [bcol, lane + j * 16, ncol], v)
        tmask = lane < 2 * (_L % 16)
        pidx = _L - (_L % 16) + (lane >> 1)
        nidx = lane & 1
        v = plsc.load_gather(num_v, [nidx, bcol, pidx], mask=tmask)
        plsc.store_scatter(np_v, [bcol, pidx, nidx], v, mask=tmask)

    @pl.loop(0, _B_PER_W)
    def _row(bb):
        b = b0 + bb
        descs = [
            pltpu.async_copy(
                tab_hbm.at[idx_v.at[e, bb]],
                g3.at[e],
                sem,
            )
            for e in range(_N_EMB)
        ]
        for d in descs:
            d.wait()
        writes = [
            pltpu.async_copy(
                g3.at[e],
                out_hbm.at[b, :, pl.ds(e * _EMB_DIM, _EMB_DIM)],
                wsem,
            )
            for e in range(_N_EMB)
        ]
        writes.append(
            pltpu.async_copy(
                np_v.at[bb], out_hbm.at[b, :, pl.ds(_EMB_COLS, _N_NUM)], wsem
            )
        )
        for w in writes:
            w.wait()


@jax.jit
def _sc_lookup(idx3, num3, tab2):
    mesh = plsc.VectorSubcoreMesh(core_axis_name="c", subcore_axis_name="s")
    return pl.kernel(
        _body,
        out_type=jax.ShapeDtypeStruct((_B, _L, _D_OUT), jnp.float32),
        mesh=mesh,
        scratch_types=[
            pltpu.VMEM((_N_EMB, _B_PER_W, _L), jnp.int32),
            pltpu.VMEM((_N_NUM, _B_PER_W, _L), jnp.float32),
            pltpu.VMEM((_B_PER_W, _L, _N_NUM), jnp.float32),
            pltpu.VMEM((_N_EMB, _L, _EMB_DIM), jnp.float32),
            pltpu.SemaphoreType.DMA,
            pltpu.SemaphoreType.DMA,
        ],
        compiler_params=pltpu.CompilerParams(
            use_tc_tiling_on_sc=False, needs_layout_passes=False
        ),
    )(idx3, num3, tab2)


def kernel(idx, numeric_feats, event_time, tables):
    idx3 = idx.astype(jnp.int32)
    num3 = numeric_feats.astype(jnp.float32)
    tabt = jnp.transpose(tables, (0, 2, 1))
    tab2 = _sc_convert(tabt).reshape(_N_EMB * _VOCAB, _EMB_DIM)
    out = _sc_lookup(idx3, num3, tab2)
    return (out, event_time)


# speedup vs baseline: 1.6278x; 1.6278x over previous
import jax
import jax.numpy as jnp
from jax import lax
from jax.experimental import pallas as pl
from jax.experimental.pallas import tpu as pltpu
from jax.experimental.pallas import tpu_sc as plsc

_N_EMB = 26
_VOCAB = 100000
_EMB_DIM = 32
_B = 1024
_L = 50
_N_NUM = 2

_EMB_COLS = _N_EMB * _EMB_DIM
_D_OUT = _EMB_COLS + _N_NUM

_NC = 2
_NS = 16
_NW = _NC * _NS
_B_PER_W = _B // _NW

def _body(idx_hbm, num_hbm, tab_hbm, out_hbm, idx_v, num_v, np_v, g3, sem, wsem):
    wid = lax.axis_index("s") * _NC + lax.axis_index("c")
    b0 = wid * _B_PER_W
    lane = jax.lax.iota(jnp.int32, 16)

    for e in range(_N_EMB):
        pltpu.sync_copy(idx_hbm.at[e, pl.ds(b0, _B_PER_W)], idx_v.at[e])
    for n in range(_N_NUM):
        pltpu.sync_copy(num_hbm.at[n, pl.ds(b0, _B_PER_W)], num_v.at[n])

    for e in range(1, _N_EMB):
        off = jnp.full((16,), e * _VOCAB, jnp.int32)

        @pl.loop(0, _B_PER_W)
        def _add(bb, e=e, off=off):
            for j in range(_L // 16):
                sl = pl.ds(j * 16, 16)
                idx_v[e, bb, sl] = idx_v[e, bb, sl] + off
            tl = pl.ds(_L - 16, 16)
            tmask = lane >= 16 - (_L % 16)
            v = plsc.load_gather(idx_v, [jnp.full((16,), e, jnp.int32),
                                         jnp.full((16,), bb, jnp.int32),
                                         lane + (_L - 16)], mask=tmask)
            plsc.store_scatter(idx_v, [jnp.full((16,), e, jnp.int32),
                                       jnp.full((16,), bb, jnp.int32),
                                       lane + (_L - 16)], v + off, mask=tmask)

    @pl.loop(0, _B_PER_W)
    def _nt(bb):
        bcol = jnp.full((16,), 0, jnp.int32) + bb
        for n in range(_N_NUM):
            ncol = jnp.full((16,), n, jnp.int32)
            for j in range(_L // 16):
                v = num_v[n, bb, pl.ds(j * 16, 16)]
                plsc.store_scatter(np_v, [bcol, lane + j * 16, ncol], v)
        tmask = lane < 2 * (_L % 16)
        pidx = _L - (_L % 16) + (lane >> 1)
        nidx = lane & 1
        v = plsc.load_gather(num_v, [nidx, bcol, pidx], mask=tmask)
        plsc.store_scatter(np_v, [bcol, pidx, nidx], v, mask=tmask)

    @pl.loop(0, _B_PER_W)
    def _row(bb):
        b = b0 + bb
        descs = [
            pltpu.async_copy(
                tab_hbm.at[idx_v.at[e, bb]],
                g3.at[e],
                sem,
            )
            for e in range(_N_EMB)
        ]
        for d in descs:
            d.wait()
        writes = [
            pltpu.async_copy(
                g3.at[e],
                out_hbm.at[b, :, pl.ds(e * _EMB_DIM, _EMB_DIM)],
                wsem,
            )
            for e in range(_N_EMB)
        ]
        writes.append(
            pltpu.async_copy(
                np_v.at[bb], out_hbm.at[b, :, pl.ds(_EMB_COLS, _N_NUM)], wsem
            )
        )
        for w in writes:
            w.wait()


@jax.jit
def _sc_lookup(idx3, num3, tab2):
    mesh = plsc.VectorSubcoreMesh(core_axis_name="c", subcore_axis_name="s")
    return pl.kernel(
        _body,
        out_type=jax.ShapeDtypeStruct((_B, _L, _D_OUT), jnp.float32),
        mesh=mesh,
        scratch_types=[
            pltpu.VMEM((_N_EMB, _B_PER_W, _L), jnp.int32),
            pltpu.VMEM((_N_NUM, _B_PER_W, _L), jnp.float32),
            pltpu.VMEM((_B_PER_W, _L, _N_NUM), jnp.float32),
            pltpu.VMEM((_N_EMB, _L, _EMB_DIM), jnp.float32),
            pltpu.SemaphoreType.DMA,
            pltpu.SemaphoreType.DMA,
        ],
        compiler_params=pltpu.CompilerParams(
            use_tc_tiling_on_sc=False, needs_layout_passes=False
        ),
    )(idx3, num3, tab2)


def kernel(idx, numeric_feats, event_time, tables):
    idx3 = idx.astype(jnp.int32)
    num3 = numeric_feats.astype(jnp.float32)
    tab2 = tables.reshape(_N_EMB * _VOCAB, _EMB_DIM)
    out = _sc_lookup(idx3, num3, tab2)
    return (out, event_time)


# speedup vs baseline: 1.6388x; 1.0067x over previous
import jax
import jax.numpy as jnp
from jax import lax
from jax.experimental import pallas as pl
from jax.experimental.pallas import tpu as pltpu
from jax.experimental.pallas import tpu_sc as plsc

_N_EMB = 26
_VOCAB = 100000
_EMB_DIM = 32
_B = 1024
_L = 50
_N_NUM = 2

_EMB_COLS = _N_EMB * _EMB_DIM
_D_OUT = _EMB_COLS + _N_NUM

_NC = 2
_NS = 16
_NW = _NC * _NS
_B_PER_W = _B // _NW
_B_HALF = _B_PER_W // 2


def _body(idx_hbm, num_hbm, tab_hbm, out_hbm,
          idx_v, num_v, np_a, np_b, g3_a, g3_b, gsem, wsem_a, wsem_b):
    wid = lax.axis_index("s") * _NC + lax.axis_index("c")
    b0 = wid * _B_PER_W
    lane = jax.lax.iota(jnp.int32, 16)

    for n in range(_N_NUM):
        pltpu.sync_copy(num_hbm.at[n, pl.ds(b0, _B_PER_W)], num_v.at[n])

    def stage_idx(half):
        for e in range(_N_EMB):
            pltpu.sync_copy(
                idx_hbm.at[e, pl.ds(b0 + half * _B_HALF, _B_HALF)],
                idx_v.at[e],
            )
        for e in range(1, _N_EMB):
            off = jnp.full((16,), e * _VOCAB, jnp.int32)

            @pl.loop(0, _B_HALF)
            def _add(bb, e=e, off=off):
                for j in range(_L // 16):
                    sl = pl.ds(j * 16, 16)
                    idx_v[e, bb, sl] = idx_v[e, bb, sl] + off
                tmask = lane >= 16 - (_L % 16)
                ecol = jnp.full((16,), e, jnp.int32)
                bcol = jnp.full((16,), 0, jnp.int32) + bb
                pcol = lane + (_L - 16)
                v = plsc.load_gather(idx_v, [ecol, bcol, pcol], mask=tmask)
                plsc.store_scatter(idx_v, [ecol, bcol, pcol], v + off,
                                   mask=tmask)

    stage_idx(0)

    def row_body(bb, g3, np_v, wsem):
        b = b0 + bb
        rows_w = [
            (lambda e=e: (g3.at[e],
                          out_hbm.at[b, :, pl.ds(e * _EMB_DIM, _EMB_DIM)]))()
            for e in range(_N_EMB)
        ]
        rows_w.append((np_v, out_hbm.at[b, :, pl.ds(_EMB_COLS, _N_NUM)]))

        @pl.when(bb >= 2)
        def _drain():
            for src, dst in rows_w:
                pltpu.make_async_copy(src, dst, wsem).wait()

        for n in range(_N_NUM):
            ncol = jnp.full((16,), n, jnp.int32)
            for j in range(_L // 16):
                v = num_v[n, bb, pl.ds(j * 16, 16)]
                plsc.store_scatter(np_v, [lane + j * 16, ncol], v)
        tmask = lane < 2 * (_L % 16)
        pidx = _L - (_L % 16) + (lane >> 1)
        nidx = lane & 1
        bcol = jnp.full((16,), 0, jnp.int32) + bb
        v = plsc.load_gather(num_v, [nidx, bcol, pidx], mask=tmask)
        plsc.store_scatter(np_v, [pidx, nidx], v, mask=tmask)

        descs = [
            pltpu.async_copy(
                tab_hbm.at[idx_v.at[e, lax.rem(bb, _B_HALF)]], g3.at[e], gsem
            )
            for e in range(_N_EMB)
        ]
        for d in descs:
            d.wait()
        for src, dst in rows_w:
            pltpu.async_copy(src, dst, wsem)

    @pl.loop(0, _B_PER_W)
    def _row(bb):
        @pl.when(bb == _B_HALF)
        def _restage():
            stage_idx(1)

        @pl.when(lax.rem(bb, 2) == 0)
        def _even():
            row_body(bb, g3_a, np_a, wsem_a)

        @pl.when(lax.rem(bb, 2) == 1)
        def _odd():
            row_body(bb, g3_b, np_b, wsem_b)

    for g3, np_v, wsem in ((g3_a, np_a, wsem_a), (g3_b, np_b, wsem_b)):
        for e in range(_N_EMB):
            pltpu.make_async_copy(
                g3.at[e], out_hbm.at[b0, :, pl.ds(e * _EMB_DIM, _EMB_DIM)],
                wsem,
            ).wait()
        pltpu.make_async_copy(
            np_v, out_hbm.at[b0, :, pl.ds(_EMB_COLS, _N_NUM)], wsem
        ).wait()


@jax.jit
def _sc_lookup(idx3, num3, tab2):
    mesh = plsc.VectorSubcoreMesh(core_axis_name="c", subcore_axis_name="s")
    return pl.kernel(
        _body,
        out_type=jax.ShapeDtypeStruct((_B, _L, _D_OUT), jnp.float32),
        mesh=mesh,
        scratch_types=[
            pltpu.VMEM((_N_EMB, _B_HALF, _L), jnp.int32),
            pltpu.VMEM((_N_NUM, _B_PER_W, _L), jnp.float32),
            pltpu.VMEM((_L, _N_NUM), jnp.float32),
            pltpu.VMEM((_L, _N_NUM), jnp.float32),
            pltpu.VMEM((_N_EMB, _L, _EMB_DIM), jnp.float32),
            pltpu.VMEM((_N_EMB, _L, _EMB_DIM), jnp.float32),
            pltpu.SemaphoreType.DMA,
            pltpu.SemaphoreType.DMA,
            pltpu.SemaphoreType.DMA,
        ],
        compiler_params=pltpu.CompilerParams(
            use_tc_tiling_on_sc=False, needs_layout_passes=False
        ),
    )(idx3, num3, tab2)


def kernel(idx, numeric_feats, event_time, tables):
    idx3 = idx.astype(jnp.int32)
    num3 = numeric_feats.astype(jnp.float32)
    tab2 = tables.reshape(_N_EMB * _VOCAB, _EMB_DIM)
    out = _sc_lookup(idx3, num3, tab2)
    return (out, event_time)
